# baseline (speedup 1.0000x reference)
import functools

import jax
import jax.numpy as jnp
from jax import lax
from jax.experimental import pallas as pl
from jax.experimental.pallas import tpu as pltpu
from jax.experimental.pallas import tpu_sc as plsc

NUM_CELLS = 200000
FEAT = 128
PRE_W = 9
ROW_W = PRE_W + 3 * FEAT
C = 64
NCHUNK = NUM_CELLS // C
NWORKERS = 32
NK = (NCHUNK + NWORKERS - 1) // NWORKERS


def _sc_assemble(u2, tri_flat, pre_flat):
    mesh = plsc.VectorSubcoreMesh(core_axis_name="c", subcore_axis_name="s")

    @functools.partial(
        pl.kernel,
        mesh=mesh,
        out_type=jax.ShapeDtypeStruct((NUM_CELLS, ROW_W), jnp.float32),
        scratch_types=[
            pltpu.VMEM((3 * C,), jnp.int32),
            pltpu.VMEM((3, C), jnp.int32),
            pltpu.VMEM((PRE_W * C,), jnp.float32),
            pltpu.VMEM((3, C, FEAT), jnp.float32),
            pltpu.VMEM((C, ROW_W), jnp.float32),
            pltpu.SemaphoreType.DMA,
        ],
        compiler_params=pltpu.CompilerParams(
            use_tc_tiling_on_sc=False, needs_layout_passes=False
        ),
    )
    def asm(u_hbm, tri_hbm, pre_hbm, out_hbm, tri_v, idx_v, pre_v, f_v, buf, sem):
        wid = lax.axis_index("s") * 2 + lax.axis_index("c")
        lane = jnp.arange(16, dtype=jnp.int32)

        def step(k, carry):
            c = wid + NWORKERS * k

            @pl.when(c < NCHUNK)
            def _():
                base = c * C
                pltpu.sync_copy(tri_hbm.at[pl.ds(base * 3, 3 * C)], tri_v)
                pltpu.sync_copy(pre_hbm.at[pl.ds(base * PRE_W, PRE_W * C)], pre_v)
                for g in range(C // 16):
                    flat_base = lane * 3 + (48 * g)
                    for v in range(3):
                        vals = plsc.load_gather(tri_v, [flat_base + v])
                        idx_v[v, pl.ds(16 * g, 16)] = vals
                cps = []
                for v in range(3):
                    cps.append(
                        pltpu.async_copy(u_hbm.at[idx_v.at[v]], f_v.at[v], sem)
                    )
                for cp in cps:
                    cp.wait()
                for g in range(C // 16):
                    rows = lane + 16 * g
                    for j in range(PRE_W):
                        vals = plsc.load_gather(pre_v, [rows * PRE_W + j])
                        plsc.store_scatter(
                            buf, [rows, jnp.full((16,), j, jnp.int32)], vals
                        )

                def cell_body(i, acc):
                    for v in range(3):
                        for j in range(FEAT // 16):
                            vec = f_v[v, i, pl.ds(16 * j, 16)]
                            buf[i, pl.ds(PRE_W + FEAT * v + 16 * j, 16)] = vec
                    return acc

                lax.fori_loop(0, C, cell_body, 0)
                pltpu.sync_copy(buf, out_hbm.at[pl.ds(base, C)])

            return carry

        lax.fori_loop(0, NK, step, 0)

    return asm(u2, tri_flat, pre_flat)


def kernel(u, t, cell_centers, cell_local_vertex_pos, triangulation):
    n_nodes, feat = u.shape[1], u.shape[2]
    ncells = triangulation.shape[0]
    u2 = u.reshape(n_nodes, feat)
    tri_flat = triangulation.astype(jnp.int32).reshape(ncells * 3)
    pre_flat = jnp.concatenate(
        [
            jnp.broadcast_to(t.reshape(1, 1), (ncells, 1)),
            cell_centers,
            cell_local_vertex_pos.reshape(ncells, 6),
        ],
        axis=1,
    ).reshape(ncells * PRE_W)
    out = _sc_assemble(u2, tri_flat, pre_flat)
    return out[None]


# speedup vs baseline: 1.1075x; 1.1075x over previous
import functools

import jax
import jax.numpy as jnp
from jax import lax
from jax.experimental import pallas as pl
from jax.experimental.pallas import tpu as pltpu
from jax.experimental.pallas import tpu_sc as plsc

NUM_CELLS = 200000
FEAT = 128
PRE_W = 9
ROW_W = PRE_W + 3 * FEAT
C = 64
NCHUNK = NUM_CELLS // C
NWORKERS = 32
NK = (NCHUNK + NWORKERS - 1) // NWORKERS


def _sc_assemble(u2, tri_flat, pre_flat):
    mesh = plsc.VectorSubcoreMesh(core_axis_name="c", subcore_axis_name="s")

    @functools.partial(
        pl.kernel,
        mesh=mesh,
        out_type=jax.ShapeDtypeStruct((NUM_CELLS * ROW_W,), jnp.float32),
        scratch_types=[
            pltpu.VMEM((3 * C,), jnp.int32),
            pltpu.VMEM((C,), jnp.int32),
            pltpu.VMEM((C,), jnp.int32),
            pltpu.VMEM((C,), jnp.int32),
            pltpu.VMEM((PRE_W * C,), jnp.float32),
            pltpu.VMEM((3, C, FEAT), jnp.float32),
            pltpu.VMEM((C * ROW_W,), jnp.float32),
            pltpu.SemaphoreType.DMA,
        ],
        compiler_params=pltpu.CompilerParams(needs_layout_passes=False),
    )
    def asm(
        u_hbm, tri_hbm, pre_hbm, out_hbm,
        tri_v, idx0, idx1, idx2, pre_v, f_v, buf, sem,
    ):
        wid = lax.axis_index("s") * 2 + lax.axis_index("c")
        lane = jnp.arange(16, dtype=jnp.int32)
        idxs = (idx0, idx1, idx2)

        def step(k, carry):
            c = wid + NWORKERS * k

            @pl.when(c < NCHUNK)
            def _():
                base = c * C
                pltpu.sync_copy(tri_hbm.at[pl.ds(base * 3, 3 * C)], tri_v)
                pltpu.sync_copy(pre_hbm.at[pl.ds(base * PRE_W, PRE_W * C)], pre_v)
                for g in range(C // 16):
                    flat_base = lane * 3 + (48 * g)
                    for v in range(3):
                        vals = plsc.load_gather(tri_v, [flat_base + v])
                        idxs[v][pl.ds(16 * g, 16)] = vals
                cps = []
                for v in range(3):
                    cps.append(
                        pltpu.async_copy(u_hbm.at[idxs[v]], f_v.at[v], sem)
                    )
                for cp in cps:
                    cp.wait()
                for g in range(C // 16):
                    rows = lane + 16 * g
                    for j in range(PRE_W):
                        vals = plsc.load_gather(pre_v, [rows * PRE_W + j])
                        plsc.store_scatter(buf, [rows * ROW_W + j], vals)

                def cell_body(i, acc):
                    row = i * ROW_W + PRE_W
                    for v in range(3):
                        for j in range(FEAT // 16):
                            vec = f_v[v, i, pl.ds(16 * j, 16)]
                            buf[pl.ds(row + FEAT * v + 16 * j, 16)] = vec
                    return acc

                lax.fori_loop(0, C, cell_body, 0)
                pltpu.sync_copy(buf, out_hbm.at[pl.ds(base * ROW_W, C * ROW_W)])

            return carry

        lax.fori_loop(0, NK, step, 0)

    return asm(u2, tri_flat, pre_flat)


def kernel(u, t, cell_centers, cell_local_vertex_pos, triangulation):
    n_nodes, feat = u.shape[1], u.shape[2]
    ncells = triangulation.shape[0]
    u2 = u.reshape(n_nodes, feat)
    tri_flat = triangulation.astype(jnp.int32).reshape(ncells * 3)
    pre_flat = jnp.concatenate(
        [
            jnp.broadcast_to(t.reshape(1, 1), (ncells, 1)),
            cell_centers,
            cell_local_vertex_pos.reshape(ncells, 6),
        ],
        axis=1,
    ).reshape(ncells * PRE_W)
    out = _sc_assemble(u2, tri_flat, pre_flat)
    return out.reshape(1, ncells, ROW_W)


# speedup vs baseline: 1.2715x; 1.1480x over previous
import functools

import jax
import jax.numpy as jnp
from jax import lax
from jax.experimental import pallas as pl
from jax.experimental.pallas import tpu as pltpu
from jax.experimental.pallas import tpu_sc as plsc

NUM_CELLS = 200000
FEAT = 128
C = 64
NCHUNK = NUM_CELLS // C
NWORKERS = 32
NK = (NCHUNK + NWORKERS - 1) // NWORKERS


def _sc_gather(u2, tri_flat):
    mesh = plsc.VectorSubcoreMesh(core_axis_name="c", subcore_axis_name="s")

    @functools.partial(
        pl.kernel,
        mesh=mesh,
        out_type=jax.ShapeDtypeStruct((NUM_CELLS, 3 * FEAT), jnp.float32),
        scratch_types=[
            pltpu.VMEM((3 * C,), jnp.int32),
            pltpu.VMEM((C,), jnp.int32),
            pltpu.VMEM((C,), jnp.int32),
            pltpu.VMEM((C,), jnp.int32),
            pltpu.VMEM((3, C, FEAT), jnp.float32),
            pltpu.SemaphoreType.DMA,
        ],
        compiler_params=pltpu.CompilerParams(
            use_tc_tiling_on_sc=False, needs_layout_passes=False
        ),
    )
    def asm(u_hbm, tri_hbm, out_hbm, tri_v, idx0, idx1, idx2, f_v, sem):
        wid = lax.axis_index("s") * 2 + lax.axis_index("c")
        lane = jnp.arange(16, dtype=jnp.int32)
        idxs = (idx0, idx1, idx2)

        def step(k, carry):
            c = wid + NWORKERS * k

            @pl.when(c < NCHUNK)
            def _():
                base = c * C
                pltpu.sync_copy(tri_hbm.at[pl.ds(base * 3, 3 * C)], tri_v)
                for g in range(C // 16):
                    flat_base = lane * 3 + (48 * g)
                    for v in range(3):
                        vals = plsc.load_gather(tri_v, [flat_base + v])
                        idxs[v][pl.ds(16 * g, 16)] = vals
                cps = []
                for v in range(3):
                    cps.append(
                        pltpu.async_copy(u_hbm.at[idxs[v]], f_v.at[v], sem)
                    )
                for cp in cps:
                    cp.wait()
                for v in range(3):
                    pltpu.sync_copy(
                        f_v.at[v],
                        out_hbm.at[pl.ds(base, C), pl.ds(FEAT * v, FEAT)],
                    )

            return carry

        lax.fori_loop(0, NK, step, 0)

    return asm(u2, tri_flat)


def kernel(u, t, cell_centers, cell_local_vertex_pos, triangulation):
    n_nodes, feat = u.shape[1], u.shape[2]
    ncells = triangulation.shape[0]
    u2 = u.reshape(n_nodes, feat)
    tri_flat = triangulation.astype(jnp.int32).reshape(ncells * 3)
    vf = _sc_gather(u2, tri_flat)
    time = jnp.broadcast_to(t.reshape(1, 1), (ncells, 1))
    vp = cell_local_vertex_pos.reshape(ncells, 6)
    out = jnp.concatenate([time, cell_centers, vp, vf], axis=1)
    return out[None]


# speedup vs baseline: 1.2838x; 1.0096x over previous
import functools

import jax
import jax.numpy as jnp
from jax import lax
from jax.experimental import pallas as pl
from jax.experimental.pallas import tpu as pltpu
from jax.experimental.pallas import tpu_sc as plsc

NUM_CELLS = 200000
FEAT = 128
PRE_W = 9
ROW_W = PRE_W + 3 * FEAT
C = 64
NCHUNK = NUM_CELLS // C
NWORKERS = 32
NK = (NCHUNK + NWORKERS - 1) // NWORKERS
CONCAT_R = 2000


def _sc_gather(u2, tri_flat):
    mesh = plsc.VectorSubcoreMesh(core_axis_name="c", subcore_axis_name="s")

    @functools.partial(
        pl.kernel,
        mesh=mesh,
        out_type=jax.ShapeDtypeStruct((NUM_CELLS, 3 * FEAT), jnp.float32),
        scratch_types=[
            pltpu.VMEM((3 * C,), jnp.int32),
            pltpu.VMEM((C,), jnp.int32),
            pltpu.VMEM((C,), jnp.int32),
            pltpu.VMEM((C,), jnp.int32),
            pltpu.VMEM((3, C, FEAT), jnp.float32),
            pltpu.SemaphoreType.DMA,
        ],
        compiler_params=pltpu.CompilerParams(
            use_tc_tiling_on_sc=False, needs_layout_passes=False
        ),
    )
    def asm(u_hbm, tri_hbm, out_hbm, tri_v, idx0, idx1, idx2, f_v, sem):
        wid = lax.axis_index("s") * 2 + lax.axis_index("c")
        lane = jnp.arange(16, dtype=jnp.int32)
        idxs = (idx0, idx1, idx2)

        def step(k, carry):
            c = wid + NWORKERS * k

            @pl.when(c < NCHUNK)
            def _():
                base = c * C
                pltpu.sync_copy(tri_hbm.at[pl.ds(base * 3, 3 * C)], tri_v)
                for g in range(C // 16):
                    flat_base = lane * 3 + (48 * g)
                    for v in range(3):
                        vals = plsc.load_gather(tri_v, [flat_base + v])
                        idxs[v][pl.ds(16 * g, 16)] = vals
                cps = []
                for v in range(3):
                    cps.append(
                        pltpu.async_copy(u_hbm.at[idxs[v]], f_v.at[v], sem)
                    )
                for cp in cps:
                    cp.wait()
                for v in range(3):
                    pltpu.sync_copy(
                        f_v.at[v],
                        out_hbm.at[pl.ds(base, C), pl.ds(FEAT * v, FEAT)],
                    )

            return carry

        lax.fori_loop(0, NK, step, 0)

    return asm(u2, tri_flat)


def _tc_concat_body(pre_ref, vf_ref, out_ref):
    out_ref[:, 0:PRE_W] = pre_ref[...]
    out_ref[:, PRE_W:ROW_W] = vf_ref[...]


def _tc_concat(pre, vf):
    grid = NUM_CELLS // CONCAT_R
    return pl.pallas_call(
        _tc_concat_body,
        grid=(grid,),
        in_specs=[
            pl.BlockSpec((CONCAT_R, PRE_W), lambda i: (i, 0)),
            pl.BlockSpec((CONCAT_R, 3 * FEAT), lambda i: (i, 0)),
        ],
        out_specs=pl.BlockSpec((CONCAT_R, ROW_W), lambda i: (i, 0)),
        out_shape=jax.ShapeDtypeStruct((NUM_CELLS, ROW_W), jnp.float32),
    )(pre, vf)


def kernel(u, t, cell_centers, cell_local_vertex_pos, triangulation):
    n_nodes, feat = u.shape[1], u.shape[2]
    ncells = triangulation.shape[0]
    u2 = u.reshape(n_nodes, feat)
    tri_flat = triangulation.astype(jnp.int32).reshape(ncells * 3)
    vf = _sc_gather(u2, tri_flat)
    pre = jnp.concatenate(
        [
            jnp.broadcast_to(t.reshape(1, 1), (ncells, 1)),
            cell_centers,
            cell_local_vertex_pos.reshape(ncells, 6),
        ],
        axis=1,
    )
    out = _tc_concat(pre, vf)
    return out[None]


# speedup vs baseline: 1.4377x; 1.1199x over previous
import functools

import jax
import jax.numpy as jnp
from jax import lax
from jax.experimental import pallas as pl
from jax.experimental.pallas import tpu as pltpu
from jax.experimental.pallas import tpu_sc as plsc

NUM_CELLS = 200000
FEAT = 128
PRE_W = 9
ROW_W = PRE_W + 3 * FEAT
C = 64
NCHUNK = NUM_CELLS // C
NWORKERS = 32
NK = (NCHUNK + NWORKERS - 1) // NWORKERS
CONCAT_R = 2000


def _sc_gather(u2, tri_flat):
    mesh = plsc.VectorSubcoreMesh(core_axis_name="c", subcore_axis_name="s")

    @functools.partial(
        pl.kernel,
        mesh=mesh,
        out_type=jax.ShapeDtypeStruct((NUM_CELLS, 3 * FEAT), jnp.float32),
        scratch_types=[
            pltpu.VMEM((3 * C,), jnp.int32),
            pltpu.VMEM((C,), jnp.int32),
            pltpu.VMEM((C,), jnp.int32),
            pltpu.VMEM((C,), jnp.int32),
            pltpu.VMEM((3, C, FEAT), jnp.float32),
            pltpu.SemaphoreType.DMA,
        ],
        compiler_params=pltpu.CompilerParams(needs_layout_passes=False),
    )
    def asm(u_hbm, tri_hbm, out_hbm, tri_v, idx0, idx1, idx2, f_v, sem):
        wid = lax.axis_index("s") * 2 + lax.axis_index("c")
        lane = jnp.arange(16, dtype=jnp.int32)
        idxs = (idx0, idx1, idx2)

        def step(k, carry):
            c = wid + NWORKERS * k

            @pl.when(c < NCHUNK)
            def _():
                base = c * C
                pltpu.sync_copy(tri_hbm.at[pl.ds(base * 3, 3 * C)], tri_v)
                for g in range(C // 16):
                    flat_base = lane * 3 + (48 * g)
                    for v in range(3):
                        vals = plsc.load_gather(tri_v, [flat_base + v])
                        idxs[v][pl.ds(16 * g, 16)] = vals
                cps = []
                for v in range(3):
                    cps.append(
                        pltpu.async_copy(u_hbm.at[idxs[v]], f_v.at[v], sem)
                    )
                for cp in cps:
                    cp.wait()
                for v in range(3):
                    pltpu.sync_copy(
                        f_v.at[v],
                        out_hbm.at[pl.ds(base, C), pl.ds(FEAT * v, FEAT)],
                    )

            return carry

        lax.fori_loop(0, NK, step, 0)

    return asm(u2, tri_flat)


def _tc_concat_body(pre_ref, vf_ref, out_ref):
    out_ref[:, 0:PRE_W] = pre_ref[...]
    out_ref[:, PRE_W:ROW_W] = vf_ref[...]


def _tc_concat(pre, vf):
    grid = NUM_CELLS // CONCAT_R
    return pl.pallas_call(
        _tc_concat_body,
        grid=(grid,),
        in_specs=[
            pl.BlockSpec((CONCAT_R, PRE_W), lambda i: (i, 0)),
            pl.BlockSpec((CONCAT_R, 3 * FEAT), lambda i: (i, 0)),
        ],
        out_specs=pl.BlockSpec((CONCAT_R, ROW_W), lambda i: (i, 0)),
        out_shape=jax.ShapeDtypeStruct((NUM_CELLS, ROW_W), jnp.float32),
    )(pre, vf)


def kernel(u, t, cell_centers, cell_local_vertex_pos, triangulation):
    n_nodes, feat = u.shape[1], u.shape[2]
    ncells = triangulation.shape[0]
    u2 = u.reshape(n_nodes, feat)
    tri_flat = triangulation.astype(jnp.int32).reshape(ncells * 3)
    vf = _sc_gather(u2, tri_flat)
    pre = jnp.concatenate(
        [
            jnp.broadcast_to(t.reshape(1, 1), (ncells, 1)),
            cell_centers,
            cell_local_vertex_pos.reshape(ncells, 6),
        ],
        axis=1,
    )
    out = _tc_concat(pre, vf)
    return out[None]


# speedup vs baseline: 1.4969x; 1.0411x over previous
import functools

import jax
import jax.numpy as jnp
from jax import lax
from jax.experimental import pallas as pl
from jax.experimental.pallas import tpu as pltpu
from jax.experimental.pallas import tpu_sc as plsc

NUM_NODES = 100000
NUM_CELLS = 200000
CPAD = 200064
FEAT = 128
PRE_W = 9
ROW_W = PRE_W + 3 * FEAT
NWORKERS = 32
NJ = (ROW_W + NWORKERS - 1) // NWORKERS
CH = 8000
NCH = NUM_CELLS // CH


def _sc_assemble_cols(ut_flat, trit_flat, pre_flat):
    mesh = plsc.VectorSubcoreMesh(core_axis_name="c", subcore_axis_name="s")

    @functools.partial(
        pl.kernel,
        mesh=mesh,
        out_type=jax.ShapeDtypeStruct((ROW_W * CPAD,), jnp.float32),
        scratch_types=[
            pltpu.VMEM((NUM_NODES,), jnp.float32),
            pltpu.VMEM((CH,), jnp.int32),
            pltpu.VMEM((CH,), jnp.float32),
        ],
        compiler_params=pltpu.CompilerParams(needs_layout_passes=False),
    )
    def asm(ut_hbm, trit_hbm, pre_hbm, out_hbm, urow_v, idx_v, out_v):
        wid = lax.axis_index("s") * 2 + lax.axis_index("c")

        def col_body(jj, carry):
            j = wid + NWORKERS * jj

            @pl.when(j < PRE_W)
            def _():
                def pchunk(kk, carry2):
                    c0 = kk * CH
                    pltpu.sync_copy(
                        pre_hbm.at[pl.ds(j * NUM_CELLS + c0, CH)], out_v
                    )
                    pltpu.sync_copy(out_v, out_hbm.at[pl.ds(j * CPAD + c0, CH)])
                    return carry2

                lax.fori_loop(0, NCH, pchunk, 0)

            @pl.when((j >= PRE_W) & (j < ROW_W))
            def _():
                g = j - PRE_W
                v = g // FEAT
                f = g % FEAT
                pltpu.sync_copy(
                    ut_hbm.at[pl.ds(f * NUM_NODES, NUM_NODES)], urow_v
                )

                def gchunk(kk, carry2):
                    c0 = kk * CH
                    pltpu.sync_copy(
                        trit_hbm.at[pl.ds(v * NUM_CELLS + c0, CH)], idx_v
                    )

                    def g16(i, carry3):
                        vec = idx_v[pl.ds(i * 16, 16)]
                        out_v[pl.ds(i * 16, 16)] = plsc.load_gather(
                            urow_v, [vec]
                        )
                        return carry3

                    lax.fori_loop(0, CH // 16, g16, 0)
                    pltpu.sync_copy(out_v, out_hbm.at[pl.ds(j * CPAD + c0, CH)])
                    return carry2

                lax.fori_loop(0, NCH, gchunk, 0)

            return carry

        lax.fori_loop(0, NJ, col_body, 0)

    return asm(ut_flat, trit_flat, pre_flat)


def kernel(u, t, cell_centers, cell_local_vertex_pos, triangulation):
    n_nodes, feat = u.shape[1], u.shape[2]
    ncells = triangulation.shape[0]
    ut_flat = jnp.transpose(u.reshape(n_nodes, feat)).reshape(n_nodes * feat)
    trit_flat = (
        jnp.transpose(triangulation.astype(jnp.int32)).reshape(3 * ncells)
    )
    vp = cell_local_vertex_pos.reshape(ncells, 6)
    pre_flat = jnp.concatenate(
        [jnp.broadcast_to(t.reshape(1, 1), (ncells, 1)), cell_centers, vp],
        axis=0 if False else 1,
    )
    pre_flat = jnp.transpose(pre_flat).reshape(PRE_W * ncells)
    full = _sc_assemble_cols(ut_flat, trit_flat, pre_flat)
    out = jnp.transpose(full.reshape(ROW_W, CPAD)[:, :ncells])
    return out[None]


# speedup vs baseline: 1.8500x; 1.2359x over previous
import functools

import jax
import jax.numpy as jnp
from jax import lax
from jax.experimental import pallas as pl
from jax.experimental.pallas import tpu as pltpu
from jax.experimental.pallas import tpu_sc as plsc

NUM_NODES = 100000
NUM_CELLS = 200000
CPAD = 200064
FEAT = 128
PRE_W = 9
ROW_W = PRE_W + 3 * FEAT
NWORKERS = 32
NJ = (ROW_W + NWORKERS - 1) // NWORKERS
CH = 4000
NCH = NUM_CELLS // CH


def _sc_assemble_cols(ut_flat, trit_flat, pre_flat):
    mesh = plsc.VectorSubcoreMesh(core_axis_name="c", subcore_axis_name="s")

    @functools.partial(
        pl.kernel,
        mesh=mesh,
        out_type=jax.ShapeDtypeStruct((ROW_W * CPAD,), jnp.float32),
        scratch_types=[
            pltpu.VMEM((NUM_NODES,), jnp.float32),
            pltpu.VMEM((CH,), jnp.int32),
            pltpu.VMEM((CH,), jnp.int32),
            pltpu.VMEM((CH,), jnp.float32),
            pltpu.VMEM((CH,), jnp.float32),
            pltpu.SemaphoreType.DMA,
            pltpu.SemaphoreType.DMA,
        ],
        compiler_params=pltpu.CompilerParams(needs_layout_passes=False),
    )
    def asm(
        ut_hbm, trit_hbm, pre_hbm, out_hbm,
        urow_v, idx0_v, idx1_v, out0_v, out1_v, idx_sem, out_sem,
    ):
        wid = lax.axis_index("s") * 2 + lax.axis_index("c")
        idx_bufs = (idx0_v, idx1_v)
        out_bufs = (out0_v, out1_v)

        def idx_copy(v, c0, b):
            return pltpu.make_async_copy(
                trit_hbm.at[pl.ds(v * NUM_CELLS + c0, CH)], idx_bufs[b], idx_sem
            )

        def out_copy(j, c0, b):
            return pltpu.make_async_copy(
                out_bufs[b], out_hbm.at[pl.ds(j * CPAD + c0, CH)], out_sem
            )

        def gather_chunk(b):
            for i in range(CH // 16):
                vec = idx_bufs[b][pl.ds(i * 16, 16)]
                out_bufs[b][pl.ds(i * 16, 16)] = plsc.load_gather(urow_v, [vec])

        def col_body(jj, carry):
            j = wid + NWORKERS * jj

            @pl.when(j < PRE_W)
            def _():
                def pchunk(kk, carry2):
                    c0 = kk * CH
                    pltpu.sync_copy(
                        pre_hbm.at[pl.ds(j * NUM_CELLS + c0, CH)], out0_v
                    )
                    pltpu.sync_copy(
                        out0_v, out_hbm.at[pl.ds(j * CPAD + c0, CH)]
                    )
                    return carry2

                lax.fori_loop(0, NCH, pchunk, 0)

            @pl.when((j >= PRE_W) & (j < ROW_W))
            def _():
                g = j - PRE_W
                v = g // FEAT
                f = g % FEAT
                pltpu.sync_copy(
                    ut_hbm.at[pl.ds(f * NUM_NODES, NUM_NODES)], urow_v
                )
                idx_copy(v, 0, 0).start()

                def pipe(kk2, carry2):
                    k0 = 2 * kk2
                    idx_copy(v, (k0 + 1) * CH, 1).start()
                    idx_copy(v, k0 * CH, 0).wait()

                    @pl.when(kk2 > 0)
                    def _():
                        out_copy(j, 0, 0).wait()

                    gather_chunk(0)
                    out_copy(j, k0 * CH, 0).start()
                    @pl.when(kk2 < NCH // 2 - 1)
                    def _():
                        idx_copy(v, (k0 + 2) * CH, 0).start()

                    idx_copy(v, (k0 + 1) * CH, 1).wait()

                    @pl.when(kk2 > 0)
                    def _():
                        out_copy(j, 0, 1).wait()

                    gather_chunk(1)
                    out_copy(j, (k0 + 1) * CH, 1).start()
                    return carry2

                lax.fori_loop(0, NCH // 2, pipe, 0)
                out_copy(j, 0, 0).wait()
                out_copy(j, 0, 1).wait()

            return carry

        lax.fori_loop(0, NJ, col_body, 0)

    return asm(ut_flat, trit_flat, pre_flat)


def kernel(u, t, cell_centers, cell_local_vertex_pos, triangulation):
    n_nodes, feat = u.shape[1], u.shape[2]
    ncells = triangulation.shape[0]
    ut_flat = jnp.transpose(u.reshape(n_nodes, feat)).reshape(n_nodes * feat)
    trit_flat = (
        jnp.transpose(triangulation.astype(jnp.int32)).reshape(3 * ncells)
    )
    vp = cell_local_vertex_pos.reshape(ncells, 6)
    pre_flat = jnp.concatenate(
        [jnp.broadcast_to(t.reshape(1, 1), (ncells, 1)), cell_centers, vp],
        axis=0 if False else 1,
    )
    pre_flat = jnp.transpose(pre_flat).reshape(PRE_W * ncells)
    full = _sc_assemble_cols(ut_flat, trit_flat, pre_flat)
    out = jnp.transpose(full.reshape(ROW_W, CPAD)[:, :ncells])
    return out[None]


# speedup vs baseline: 2.0867x; 1.1280x over previous
import functools

import jax
import jax.numpy as jnp
from jax import lax
from jax.experimental import pallas as pl
from jax.experimental.pallas import tpu as pltpu
from jax.experimental.pallas import tpu_sc as plsc

NUM_NODES = 100000
NUM_CELLS = 200000
CPAD = 200000
FEAT = 128
PRE_W = 9
ROW_W = PRE_W + 3 * FEAT
NWORKERS = 32
NJ = (ROW_W + NWORKERS - 1) // NWORKERS
CH = 4000
NCH = NUM_CELLS // CH


def _sc_assemble_cols(ut_flat, trit_flat, pre_flat):
    mesh = plsc.VectorSubcoreMesh(core_axis_name="c", subcore_axis_name="s")

    @functools.partial(
        pl.kernel,
        mesh=mesh,
        out_type=jax.ShapeDtypeStruct((ROW_W * CPAD,), jnp.float32),
        scratch_types=[
            pltpu.VMEM((NUM_NODES,), jnp.float32),
            pltpu.VMEM((CH,), jnp.int32),
            pltpu.VMEM((CH,), jnp.int32),
            pltpu.VMEM((CH,), jnp.float32),
            pltpu.VMEM((CH,), jnp.float32),
            pltpu.SemaphoreType.DMA,
            pltpu.SemaphoreType.DMA,
        ],
        compiler_params=pltpu.CompilerParams(needs_layout_passes=False),
    )
    def asm(
        ut_hbm, trit_hbm, pre_hbm, out_hbm,
        urow_v, idx0_v, idx1_v, out0_v, out1_v, idx_sem, out_sem,
    ):
        wid = lax.axis_index("s") * 2 + lax.axis_index("c")
        idx_bufs = (idx0_v, idx1_v)
        out_bufs = (out0_v, out1_v)

        def idx_copy(v, c0, b):
            return pltpu.make_async_copy(
                trit_hbm.at[pl.ds(v * NUM_CELLS + c0, CH)], idx_bufs[b], idx_sem
            )

        def out_copy(j, c0, b):
            return pltpu.make_async_copy(
                out_bufs[b], out_hbm.at[pl.ds(j * CPAD + c0, CH)], out_sem
            )

        def gather_chunk(b):
            for i in range(CH // 16):
                vec = idx_bufs[b][pl.ds(i * 16, 16)]
                out_bufs[b][pl.ds(i * 16, 16)] = plsc.load_gather(urow_v, [vec])

        def col_body(jj, carry):
            j = wid + NWORKERS * jj

            @pl.when(j < PRE_W)
            def _():
                def pchunk(kk, carry2):
                    c0 = kk * CH
                    pltpu.sync_copy(
                        pre_hbm.at[pl.ds(j * NUM_CELLS + c0, CH)], out0_v
                    )
                    pltpu.sync_copy(
                        out0_v, out_hbm.at[pl.ds(j * CPAD + c0, CH)]
                    )
                    return carry2

                lax.fori_loop(0, NCH, pchunk, 0)

            @pl.when((j >= PRE_W) & (j < ROW_W))
            def _():
                g = j - PRE_W
                v = g // FEAT
                f = g % FEAT
                pltpu.sync_copy(
                    ut_hbm.at[pl.ds(f * NUM_NODES, NUM_NODES)], urow_v
                )
                idx_copy(v, 0, 0).start()

                def pipe(kk2, carry2):
                    k0 = 2 * kk2
                    idx_copy(v, (k0 + 1) * CH, 1).start()
                    idx_copy(v, k0 * CH, 0).wait()

                    @pl.when(kk2 > 0)
                    def _():
                        out_copy(j, 0, 0).wait()

                    gather_chunk(0)
                    out_copy(j, k0 * CH, 0).start()
                    @pl.when(kk2 < NCH // 2 - 1)
                    def _():
                        idx_copy(v, (k0 + 2) * CH, 0).start()

                    idx_copy(v, (k0 + 1) * CH, 1).wait()

                    @pl.when(kk2 > 0)
                    def _():
                        out_copy(j, 0, 1).wait()

                    gather_chunk(1)
                    out_copy(j, (k0 + 1) * CH, 1).start()
                    return carry2

                lax.fori_loop(0, NCH // 2, pipe, 0)
                out_copy(j, 0, 0).wait()
                out_copy(j, 0, 1).wait()

            return carry

        lax.fori_loop(0, NJ, col_body, 0)

    return asm(ut_flat, trit_flat, pre_flat)


def kernel(u, t, cell_centers, cell_local_vertex_pos, triangulation):
    n_nodes, feat = u.shape[1], u.shape[2]
    ncells = triangulation.shape[0]
    ut_flat = jnp.transpose(u.reshape(n_nodes, feat)).reshape(n_nodes * feat)
    trit_flat = (
        jnp.transpose(triangulation.astype(jnp.int32)).reshape(3 * ncells)
    )
    vp = cell_local_vertex_pos.reshape(ncells, 6)
    pre_flat = jnp.concatenate(
        [jnp.broadcast_to(t.reshape(1, 1), (ncells, 1)), cell_centers, vp],
        axis=0 if False else 1,
    )
    pre_flat = jnp.transpose(pre_flat).reshape(PRE_W * ncells)
    full = _sc_assemble_cols(ut_flat, trit_flat, pre_flat)
    out = jnp.transpose(full.reshape(ROW_W, CPAD))
    return out[None]


# speedup vs baseline: 2.1031x; 1.0079x over previous
import functools

import jax
import jax.numpy as jnp
from jax import lax
from jax.experimental import pallas as pl
from jax.experimental.pallas import tpu as pltpu
from jax.experimental.pallas import tpu_sc as plsc

NUM_NODES = 100000
NUM_CELLS = 200000
CPAD = 200000
FEAT = 128
PRE_W = 9
ROW_W = PRE_W + 3 * FEAT
NWORKERS = 32
NJ = (ROW_W + NWORKERS - 1) // NWORKERS
CH = 4000
NCH = NUM_CELLS // CH


def _sc_assemble_cols(ut_flat, trit_flat, pre_flat):
    mesh = plsc.VectorSubcoreMesh(core_axis_name="c", subcore_axis_name="s")

    @functools.partial(
        pl.kernel,
        mesh=mesh,
        out_type=jax.ShapeDtypeStruct((ROW_W * CPAD,), jnp.float32),
        scratch_types=[
            pltpu.VMEM((NUM_NODES,), jnp.float32),
            pltpu.VMEM((CH,), jnp.int32),
            pltpu.VMEM((CH,), jnp.int32),
            pltpu.VMEM((CH,), jnp.float32),
            pltpu.VMEM((CH,), jnp.float32),
            pltpu.SemaphoreType.DMA,
            pltpu.SemaphoreType.DMA,
        ],
        compiler_params=pltpu.CompilerParams(needs_layout_passes=False),
    )
    def asm(
        ut_hbm, trit_hbm, pre_hbm, out_hbm,
        urow_v, idx0_v, idx1_v, out0_v, out1_v, idx_sem, out_sem,
    ):
        wid = lax.axis_index("s") * 2 + lax.axis_index("c")
        idx_bufs = (idx0_v, idx1_v)
        out_bufs = (out0_v, out1_v)

        def idx_copy(v, c0, b):
            return pltpu.make_async_copy(
                trit_hbm.at[pl.ds(v * NUM_CELLS + c0, CH)], idx_bufs[b], idx_sem
            )

        def out_copy(j, c0, b):
            return pltpu.make_async_copy(
                out_bufs[b], out_hbm.at[pl.ds(j * CPAD + c0, CH)], out_sem
            )

        def gather_chunk(b):
            for i in range(CH // 16):
                vec = idx_bufs[b][pl.ds(i * 16, 16)]
                out_bufs[b][pl.ds(i * 16, 16)] = plsc.load_gather(urow_v, [vec])

        def col_body(jj, carry):
            j = wid + NWORKERS * jj

            @pl.when(j < PRE_W)
            def _():
                def pre_read(c0, b):
                    return pltpu.make_async_copy(
                        pre_hbm.at[pl.ds(j * NUM_CELLS + c0, CH)],
                        out_bufs[b],
                        idx_sem,
                    )

                pre_read(0, 0).start()
                pre_read(CH, 1).start()

                def pchunk(kk2, carry2):
                    k0 = 2 * kk2
                    pre_read(k0 * CH, 0).wait()
                    out_copy(j, k0 * CH, 0).start()
                    pre_read((k0 + 1) * CH, 1).wait()
                    out_copy(j, (k0 + 1) * CH, 1).start()

                    @pl.when(kk2 < NCH // 2 - 1)
                    def _():
                        out_copy(j, 0, 0).wait()
                        pre_read((k0 + 2) * CH, 0).start()
                        out_copy(j, 0, 1).wait()
                        pre_read((k0 + 3) * CH, 1).start()

                    return carry2

                lax.fori_loop(0, NCH // 2, pchunk, 0)
                out_copy(j, 0, 0).wait()
                out_copy(j, 0, 1).wait()

            @pl.when((j >= PRE_W) & (j < ROW_W))
            def _():
                g = j - PRE_W
                v = g // FEAT
                f = g % FEAT
                pltpu.sync_copy(
                    ut_hbm.at[pl.ds(f * NUM_NODES, NUM_NODES)], urow_v
                )
                idx_copy(v, 0, 0).start()

                def pipe(kk2, carry2):
                    k0 = 2 * kk2
                    idx_copy(v, (k0 + 1) * CH, 1).start()
                    idx_copy(v, k0 * CH, 0).wait()

                    @pl.when(kk2 > 0)
                    def _():
                        out_copy(j, 0, 0).wait()

                    gather_chunk(0)
                    out_copy(j, k0 * CH, 0).start()
                    @pl.when(kk2 < NCH // 2 - 1)
                    def _():
                        idx_copy(v, (k0 + 2) * CH, 0).start()

                    idx_copy(v, (k0 + 1) * CH, 1).wait()

                    @pl.when(kk2 > 0)
                    def _():
                        out_copy(j, 0, 1).wait()

                    gather_chunk(1)
                    out_copy(j, (k0 + 1) * CH, 1).start()
                    return carry2

                lax.fori_loop(0, NCH // 2, pipe, 0)
                out_copy(j, 0, 0).wait()
                out_copy(j, 0, 1).wait()

            return carry

        lax.fori_loop(0, NJ, col_body, 0)

    return asm(ut_flat, trit_flat, pre_flat)


def kernel(u, t, cell_centers, cell_local_vertex_pos, triangulation):
    n_nodes, feat = u.shape[1], u.shape[2]
    ncells = triangulation.shape[0]
    ut_flat = jnp.transpose(u.reshape(n_nodes, feat)).reshape(n_nodes * feat)
    trit_flat = (
        jnp.transpose(triangulation.astype(jnp.int32)).reshape(3 * ncells)
    )
    vp = cell_local_vertex_pos.reshape(ncells, 6)
    pre_flat = jnp.concatenate(
        [jnp.broadcast_to(t.reshape(1, 1), (ncells, 1)), cell_centers, vp],
        axis=0 if False else 1,
    )
    pre_flat = jnp.transpose(pre_flat).reshape(PRE_W * ncells)
    full = _sc_assemble_cols(ut_flat, trit_flat, pre_flat)
    out = jnp.transpose(full.reshape(ROW_W, CPAD))
    return out[None]


# speedup vs baseline: 2.1036x; 1.0002x over previous
import functools

import jax
import jax.numpy as jnp
from jax import lax
from jax.experimental import pallas as pl
from jax.experimental.pallas import tpu as pltpu
from jax.experimental.pallas import tpu_sc as plsc

NUM_NODES = 100000
NUM_CELLS = 200000
CPAD = 200000
FEAT = 128
PRE_W = 9
ROW_W = PRE_W + 3 * FEAT
NWORKERS = 32
NJ = (ROW_W + NWORKERS - 1) // NWORKERS
CH = 4000
NCH = NUM_CELLS // CH


def _sc_assemble_cols(ut_flat, trit_flat, pre_flat):
    mesh = plsc.VectorSubcoreMesh(core_axis_name="c", subcore_axis_name="s")

    @functools.partial(
        pl.kernel,
        mesh=mesh,
        out_type=jax.ShapeDtypeStruct((ROW_W * CPAD,), jnp.float32),
        scratch_types=[
            pltpu.VMEM((NUM_NODES,), jnp.float32),
            pltpu.VMEM((CH,), jnp.int32),
            pltpu.VMEM((CH,), jnp.int32),
            pltpu.VMEM((CH,), jnp.float32),
            pltpu.VMEM((CH,), jnp.float32),
            pltpu.SemaphoreType.DMA,
            pltpu.SemaphoreType.DMA,
        ],
        compiler_params=pltpu.CompilerParams(needs_layout_passes=False),
    )
    def asm(
        ut_hbm, trit_hbm, pre_hbm, out_hbm,
        urow_v, idx0_v, idx1_v, out0_v, out1_v, idx_sem, out_sem,
    ):
        wid = lax.axis_index("s") * 2 + lax.axis_index("c")
        idx_bufs = (idx0_v, idx1_v)
        out_bufs = (out0_v, out1_v)

        def idx_copy(v, c0, b):
            return pltpu.make_async_copy(
                trit_hbm.at[pl.ds(v * NUM_CELLS + c0, CH)], idx_bufs[b], idx_sem
            )

        def out_copy(j, c0, b):
            return pltpu.make_async_copy(
                out_bufs[b], out_hbm.at[pl.ds(j * CPAD + c0, CH)], out_sem
            )

        def gather_chunk(b):
            for i in range(CH // 16):
                vec = idx_bufs[b][pl.ds(i * 16, 16)]
                out_bufs[b][pl.ds(i * 16, 16)] = plsc.load_gather(urow_v, [vec])

        def col_body(jj, carry):
            j = wid + NWORKERS * jj

            @pl.when(j < PRE_W)
            def _():
                def pre_read(c0, b):
                    return pltpu.make_async_copy(
                        pre_hbm.at[pl.ds(j * NUM_CELLS + c0, CH)],
                        out_bufs[b],
                        idx_sem,
                    )

                pre_read(0, 0).start()
                pre_read(CH, 1).start()

                def pchunk(kk2, carry2):
                    k0 = 2 * kk2
                    pre_read(k0 * CH, 0).wait()
                    out_copy(j, k0 * CH, 0).start()
                    pre_read((k0 + 1) * CH, 1).wait()
                    out_copy(j, (k0 + 1) * CH, 1).start()

                    @pl.when(kk2 < NCH // 2 - 1)
                    def _():
                        out_copy(j, 0, 0).wait()
                        pre_read((k0 + 2) * CH, 0).start()
                        out_copy(j, 0, 1).wait()
                        pre_read((k0 + 3) * CH, 1).start()

                    return carry2

                lax.fori_loop(0, NCH // 2, pchunk, 0)
                out_copy(j, 0, 0).wait()
                out_copy(j, 0, 1).wait()

            @pl.when((j >= PRE_W) & (j < ROW_W))
            def _():
                g = j - PRE_W
                v = g // FEAT
                f = g % FEAT
                pltpu.sync_copy(
                    ut_hbm.at[pl.ds(f * NUM_NODES, NUM_NODES)], urow_v
                )
                idx_copy(v, 0, 0).start()

                def pipe(kk2, carry2):
                    k0 = 2 * kk2
                    idx_copy(v, (k0 + 1) * CH, 1).start()
                    idx_copy(v, k0 * CH, 0).wait()

                    @pl.when(kk2 > 0)
                    def _():
                        out_copy(j, 0, 0).wait()

                    gather_chunk(0)
                    out_copy(j, k0 * CH, 0).start()
                    @pl.when(kk2 < NCH // 2 - 1)
                    def _():
                        idx_copy(v, (k0 + 2) * CH, 0).start()

                    idx_copy(v, (k0 + 1) * CH, 1).wait()

                    @pl.when(kk2 > 0)
                    def _():
                        out_copy(j, 0, 1).wait()

                    gather_chunk(1)
                    out_copy(j, (k0 + 1) * CH, 1).start()
                    return carry2

                lax.fori_loop(0, NCH // 2, pipe, 0)
                out_copy(j, 0, 0).wait()
                out_copy(j, 0, 1).wait()

            return carry

        lax.fori_loop(0, NJ, col_body, 0)

    return asm(ut_flat, trit_flat, pre_flat)


def kernel(u, t, cell_centers, cell_local_vertex_pos, triangulation):
    n_nodes, feat = u.shape[1], u.shape[2]
    ncells = triangulation.shape[0]
    ut_flat = jnp.transpose(u.reshape(n_nodes, feat)).reshape(n_nodes * feat)
    trit_flat = (
        jnp.transpose(triangulation.astype(jnp.int32)).reshape(3 * ncells)
    )
    vp = cell_local_vertex_pos.reshape(ncells, 6)
    pre_flat = jnp.concatenate(
        [jnp.broadcast_to(t.reshape(1, 1), (ncells, 1)), cell_centers, vp],
        axis=1,
    )
    pre_flat = jnp.transpose(pre_flat).reshape(PRE_W * ncells)
    full = _sc_assemble_cols(ut_flat, trit_flat, pre_flat)
    out = jnp.transpose(full.reshape(ROW_W, CPAD))
    return out[None]
